# baseline (speedup 1.0000x reference)
import functools

import jax
import jax.numpy as jnp
from jax import lax
from jax.experimental import pallas as pl
from jax.experimental.pallas import tpu as pltpu
from jax.experimental.pallas import tpu_sc as plsc

N = 100000
N_SPECIES = 119
L = 16
NC = 2
NS = 16
NW = NC * NS
P = 3136
H0, H1 = 1600, 1536
LAST = N - (NW - 1) * P
G0, G1 = 1408, LAST - 1408
G1_PAD = 1408

_mesh = plsc.VectorSubcoreMesh(core_axis_name="c", subcore_axis_name="s")


@functools.partial(
    pl.kernel,
    mesh=_mesh,
    out_type=jax.ShapeDtypeStruct((N,), jnp.float32),
    scratch_types=[
        pltpu.VMEM((P,), jnp.float32),
        pltpu.VMEM((P,), jnp.int32),
        pltpu.VMEM((P,), jnp.float32),
        pltpu.VMEM((N_SPECIES,), jnp.float32),
        pltpu.SemaphoreType.DMA,
        pltpu.SemaphoreType.DMA,
        pltpu.SemaphoreType.DMA,
    ],
    compiler_params=pltpu.CompilerParams(
        needs_layout_passes=False,
        disable_bounds_checks=True,
        disable_semaphore_checks=True,
    ),
)
def _per_species_offset(x_hbm, z_hbm, off_hbm, out_hbm, x_v, z_v, o_v, tab_v,
                        sem0, sem1, semo):
    wid = lax.axis_index("s") * NC + lax.axis_index("c")
    base = wid * P
    is_last = wid == NW - 1

    tab_cp = pltpu.async_copy(off_hbm, tab_v, sem0)

    def fire(a, b):
        x0 = pltpu.async_copy(x_hbm.at[pl.ds(base, a)],
                              x_v.at[pl.ds(0, a)], sem0)
        z0 = pltpu.async_copy(z_hbm.at[pl.ds(base, a)],
                              z_v.at[pl.ds(0, a)], sem0)
        x1 = pltpu.async_copy(x_hbm.at[pl.ds(base + a, b)],
                              x_v.at[pl.ds(a, b)], sem1)
        z1 = pltpu.async_copy(z_hbm.at[pl.ds(base + a, b)],
                              z_v.at[pl.ds(a, b)], sem1)
        return x0, z0, x1, z1

    @pl.when(jnp.logical_not(is_last))
    def _():
        x0, z0, x1, z1 = fire(H0, H1)
        x0.wait(); z0.wait()

    @pl.when(is_last)
    def _():
        x0, z0, x1, z1 = fire(G0, G1)
        x0.wait(); z0.wait()

    tab_cp.wait()

    n0 = jnp.where(is_last, G0, H0)

    @plsc.parallel_loop(0, n0, L, unroll=4)
    def _(s):
        o_v[pl.ds(s, L)] = x_v[pl.ds(s, L)] + plsc.load_gather(
            tab_v, [z_v[pl.ds(s, L)]])

    @pl.when(jnp.logical_not(is_last))
    def _():
        o0 = pltpu.async_copy(o_v.at[pl.ds(0, H0)],
                              out_hbm.at[pl.ds(base, H0)], semo)
        pltpu.make_async_copy(x_hbm.at[pl.ds(base + H0, H1)],
                              x_v.at[pl.ds(H0, H1)], sem1).wait()
        pltpu.make_async_copy(z_hbm.at[pl.ds(base + H0, H1)],
                              z_v.at[pl.ds(H0, H1)], sem1).wait()

        @plsc.parallel_loop(H0, P, L, unroll=4)
        def _(s):
            o_v[pl.ds(s, L)] = x_v[pl.ds(s, L)] + plsc.load_gather(
                tab_v, [z_v[pl.ds(s, L)]])

        pltpu.sync_copy(o_v.at[pl.ds(H0, H1)],
                        out_hbm.at[pl.ds(base + H0, H1)])
        o0.wait()

    @pl.when(is_last)
    def _():
        o0 = pltpu.async_copy(o_v.at[pl.ds(0, G0)],
                              out_hbm.at[pl.ds(base, G0)], semo)
        pltpu.make_async_copy(x_hbm.at[pl.ds(base + G0, G1)],
                              x_v.at[pl.ds(G0, G1)], sem1).wait()
        pltpu.make_async_copy(z_hbm.at[pl.ds(base + G0, G1)],
                              z_v.at[pl.ds(G0, G1)], sem1).wait()
        zeros = jnp.zeros((L,), jnp.int32)
        z_v[pl.ds(LAST, L)] = zeros
        z_v[pl.ds(LAST + L, L)] = zeros

        @plsc.parallel_loop(G0, G0 + G1_PAD, L, unroll=4)
        def _(s):
            o_v[pl.ds(s, L)] = x_v[pl.ds(s, L)] + plsc.load_gather(
                tab_v, [z_v[pl.ds(s, L)]])

        pltpu.sync_copy(o_v.at[pl.ds(G0, G1)],
                        out_hbm.at[pl.ds(base + G0, G1)])
        o0.wait()


def kernel(x, Z, offsets):
    return _per_species_offset(x, Z.astype(jnp.int32), offsets)


# speedup vs baseline: 1.0133x; 1.0133x over previous
import functools

import jax
import jax.numpy as jnp
from jax import lax
from jax.experimental import pallas as pl
from jax.experimental.pallas import tpu as pltpu
from jax.experimental.pallas import tpu_sc as plsc

N = 100000
N_SPECIES = 119
L = 16
NC = 2
NS = 16
NW = NC * NS
P = 3136
LAST = N - (NW - 1) * P
LAST_PAD = 2816

_mesh = plsc.VectorSubcoreMesh(core_axis_name="c", subcore_axis_name="s")


@functools.partial(
    pl.kernel,
    mesh=_mesh,
    out_type=jax.ShapeDtypeStruct((N,), jnp.float32),
    scratch_types=[
        pltpu.VMEM((P,), jnp.float32),
        pltpu.VMEM((P,), jnp.int32),
        pltpu.VMEM((P,), jnp.float32),
        pltpu.VMEM((N_SPECIES,), jnp.float32),
        pltpu.SemaphoreType.DMA,
    ],
    compiler_params=pltpu.CompilerParams(
        needs_layout_passes=False,
        disable_bounds_checks=True,
        disable_semaphore_checks=True,
    ),
)
def _per_species_offset(x_hbm, z_hbm, off_hbm, out_hbm, x_v, z_v, o_v, tab_v,
                        sem):
    wid = lax.axis_index("s") * NC + lax.axis_index("c")
    base = wid * P
    is_last = wid == NW - 1

    tab_cp = pltpu.async_copy(off_hbm, tab_v, sem)

    @pl.when(jnp.logical_not(is_last))
    def _():
        x_cp = pltpu.async_copy(x_hbm.at[pl.ds(base, P)], x_v, sem)
        z_cp = pltpu.async_copy(z_hbm.at[pl.ds(base, P)], z_v, sem)
        x_cp.wait()
        z_cp.wait()

    @pl.when(is_last)
    def _():
        x_cp = pltpu.async_copy(x_hbm.at[pl.ds(base, LAST)],
                                x_v.at[pl.ds(0, LAST)], sem)
        z_cp = pltpu.async_copy(z_hbm.at[pl.ds(base, LAST)],
                                z_v.at[pl.ds(0, LAST)], sem)
        x_cp.wait()
        z_cp.wait()
        zeros = jnp.zeros((L,), jnp.int32)
        z_v[pl.ds(LAST, L)] = zeros
        z_v[pl.ds(LAST + L, L)] = zeros

    tab_cp.wait()

    n_elems = jnp.where(is_last, LAST_PAD, P)

    @plsc.parallel_loop(0, n_elems, L, unroll=4)
    def _(s):
        o_v[pl.ds(s, L)] = x_v[pl.ds(s, L)] + plsc.load_gather(
            tab_v, [z_v[pl.ds(s, L)]])

    @pl.when(jnp.logical_not(is_last))
    def _():
        pltpu.sync_copy(o_v, out_hbm.at[pl.ds(base, P)])

    @pl.when(is_last)
    def _():
        pltpu.sync_copy(o_v.at[pl.ds(0, LAST)], out_hbm.at[pl.ds(base, LAST)])


def kernel(x, Z, offsets):
    return _per_species_offset(x, Z.astype(jnp.int32), offsets)


# speedup vs baseline: 1.0174x; 1.0041x over previous
import functools

import jax
import jax.numpy as jnp
from jax import lax
from jax.experimental import pallas as pl
from jax.experimental.pallas import tpu as pltpu
from jax.experimental.pallas import tpu_sc as plsc

N = 100000
N_SPECIES = 119
L = 16
NC = 2
NS = 16
NW = NC * NS
P = 3136

_mesh = plsc.VectorSubcoreMesh(core_axis_name="c", subcore_axis_name="s")


@functools.partial(
    pl.kernel,
    mesh=_mesh,
    out_type=jax.ShapeDtypeStruct((N,), jnp.float32),
    scratch_types=[
        pltpu.VMEM((P,), jnp.float32),
        pltpu.VMEM((P,), jnp.int32),
        pltpu.VMEM((P,), jnp.float32),
        pltpu.VMEM((N_SPECIES,), jnp.float32),
        pltpu.SemaphoreType.DMA,
    ],
    compiler_params=pltpu.CompilerParams(
        needs_layout_passes=False,
        disable_bounds_checks=True,
        disable_semaphore_checks=True,
    ),
)
def _per_species_offset(x_hbm, z_hbm, off_hbm, out_hbm, x_v, z_v, o_v, tab_v,
                        sem):
    wid = lax.axis_index("s") * NC + lax.axis_index("c")
    base = jnp.minimum(wid * P, N - P)

    tab_cp = pltpu.async_copy(off_hbm, tab_v, sem)
    x_cp = pltpu.async_copy(x_hbm.at[pl.ds(base, P)], x_v, sem)
    z_cp = pltpu.async_copy(z_hbm.at[pl.ds(base, P)], z_v, sem)
    tab_cp.wait()
    x_cp.wait()
    z_cp.wait()

    @plsc.parallel_loop(0, P, L, unroll=4)
    def _(s):
        o_v[pl.ds(s, L)] = x_v[pl.ds(s, L)] + plsc.load_gather(
            tab_v, [z_v[pl.ds(s, L)]])

    pltpu.sync_copy(o_v, out_hbm.at[pl.ds(base, P)])


def kernel(x, Z, offsets):
    return _per_species_offset(x, Z.astype(jnp.int32), offsets)


# speedup vs baseline: 1.0224x; 1.0049x over previous
import functools

import jax
import jax.numpy as jnp
from jax import lax
from jax.experimental import pallas as pl
from jax.experimental.pallas import tpu as pltpu
from jax.experimental.pallas import tpu_sc as plsc

N = 100000
N_SPECIES = 119
L = 16
NC = 2
NS = 16
NW = NC * NS
P = 3136

_mesh = plsc.VectorSubcoreMesh(core_axis_name="c", subcore_axis_name="s")


@functools.partial(
    pl.kernel,
    mesh=_mesh,
    out_type=jax.ShapeDtypeStruct((N,), jnp.float32),
    scratch_types=[
        pltpu.VMEM((P,), jnp.float32),
        pltpu.VMEM((P,), jnp.int32),
        pltpu.VMEM((P,), jnp.float32),
        pltpu.VMEM((N_SPECIES,), jnp.float32),
        pltpu.SemaphoreType.DMA,
    ],
    compiler_params=pltpu.CompilerParams(
        needs_layout_passes=False,
        disable_bounds_checks=True,
        disable_semaphore_checks=True,
        skip_device_barrier=True,
    ),
)
def _per_species_offset(x_hbm, z_hbm, off_hbm, out_hbm, x_v, z_v, o_v, tab_v,
                        sem):
    wid = lax.axis_index("s") * NC + lax.axis_index("c")
    base = jnp.minimum(wid * P, N - P)

    tab_cp = pltpu.async_copy(off_hbm, tab_v, sem)
    x_cp = pltpu.async_copy(x_hbm.at[pl.ds(base, P)], x_v, sem)
    z_cp = pltpu.async_copy(z_hbm.at[pl.ds(base, P)], z_v, sem)
    tab_cp.wait()
    x_cp.wait()
    z_cp.wait()

    @plsc.parallel_loop(0, P, L, unroll=4)
    def _(s):
        o_v[pl.ds(s, L)] = x_v[pl.ds(s, L)] + plsc.load_gather(
            tab_v, [z_v[pl.ds(s, L)]])

    pltpu.sync_copy(o_v, out_hbm.at[pl.ds(base, P)])


def kernel(x, Z, offsets):
    return _per_species_offset(x, Z.astype(jnp.int32), offsets)


# speedup vs baseline: 1.0780x; 1.0544x over previous
import functools

import jax
import jax.numpy as jnp
from jax import lax
from jax.experimental import pallas as pl
from jax.experimental.pallas import tpu as pltpu
from jax.experimental.pallas import tpu_sc as plsc

N = 100000
N_SPECIES = 119
L = 16
NC = 1
NS = 16
NW = NC * NS
P = 6272

_mesh = plsc.VectorSubcoreMesh(core_axis_name="c", subcore_axis_name="s", num_cores=1)


@functools.partial(
    pl.kernel,
    mesh=_mesh,
    out_type=jax.ShapeDtypeStruct((N,), jnp.float32),
    scratch_types=[
        pltpu.VMEM((P,), jnp.float32),
        pltpu.VMEM((P,), jnp.int32),
        pltpu.VMEM((P,), jnp.float32),
        pltpu.VMEM((N_SPECIES,), jnp.float32),
        pltpu.SemaphoreType.DMA,
    ],
    compiler_params=pltpu.CompilerParams(
        needs_layout_passes=False,
        disable_bounds_checks=True,
        disable_semaphore_checks=True,
        skip_device_barrier=True,
    ),
)
def _per_species_offset(x_hbm, z_hbm, off_hbm, out_hbm, x_v, z_v, o_v, tab_v,
                        sem):
    wid = lax.axis_index("s") * NC + lax.axis_index("c")
    base = jnp.minimum(wid * P, N - P)

    tab_cp = pltpu.async_copy(off_hbm, tab_v, sem)
    x_cp = pltpu.async_copy(x_hbm.at[pl.ds(base, P)], x_v, sem)
    z_cp = pltpu.async_copy(z_hbm.at[pl.ds(base, P)], z_v, sem)
    tab_cp.wait()
    x_cp.wait()
    z_cp.wait()

    @plsc.parallel_loop(0, P, L, unroll=4)
    def _(s):
        o_v[pl.ds(s, L)] = x_v[pl.ds(s, L)] + plsc.load_gather(
            tab_v, [z_v[pl.ds(s, L)]])

    pltpu.sync_copy(o_v, out_hbm.at[pl.ds(base, P)])


def kernel(x, Z, offsets):
    return _per_species_offset(x, Z.astype(jnp.int32), offsets)
